# baseline (speedup 1.0000x reference)
import functools

import jax
import jax.numpy as jnp
from jax import lax
from jax.experimental import pallas as pl
from jax.experimental.pallas import tpu as pltpu
from jax.experimental.pallas import tpu_sc as plsc

LB_POS = 0.9
LB_NEG = 0.005
LB_IGNORE = 255

B = 1024
C = 100000

CW = 2048
SC_CHUNKS = 21
CS = SC_CHUNKS * CW
K_TC = (C - CS + CW - 1) // CW

NW = 32
RPW = B // NW
GPW = RPW // 8
NBUF = 3
NVREG = CW // 16

_mesh = plsc.VectorSubcoreMesh(core_axis_name="c", subcore_axis_name="s")

_NEG = float("-inf")


@functools.partial(
    pl.kernel,
    mesh=_mesh,
    out_type=[jax.ShapeDtypeStruct((B * 16,), jnp.float32)] * 4,
    scratch_types=[pltpu.VMEM((8, CW), jnp.float32)] * NBUF
    + [
        pltpu.VMEM((RPW,), jnp.int32),
        pltpu.VMEM((128,), jnp.float32),
        pltpu.VMEM((128,), jnp.float32),
        pltpu.VMEM((128,), jnp.float32),
        pltpu.VMEM((128,), jnp.float32),
    ]
    + [pltpu.SemaphoreType.DMA] * (NBUF + 1),
)
def _sc_stats(
    logits_hbm,
    label_hbm,
    m_out,
    s_out,
    t_out,
    g_out,
    b0,
    b1,
    b2,
    lab_v,
    acc_m,
    acc_s,
    acc_t,
    acc_g,
    s0,
    s1,
    s2,
    sl,
):
    bufs = (b0, b1, b2)
    sems = (s0, s1, s2)
    wid = lax.axis_index("s") * 2 + lax.axis_index("c")
    row0 = wid * RPW
    iota16 = lax.iota(jnp.int32, 16)

    def src(g, l):
        return logits_hbm.at[pl.ds(row0 + g * 8, 8), pl.ds(l * CW, CW)]

    pltpu.async_copy(label_hbm.at[pl.ds(row0, RPW)], lab_v, sl)
    for b in range(NBUF):
        pltpu.async_copy(src(0, b), bufs[b], sems[b])
    pltpu.make_async_copy(label_hbm.at[pl.ds(row0, RPW)], lab_v, sl).wait()
    lab_lo = lab_v[pl.ds(0, 16)]
    lab_hi = lab_v[pl.ds(16, 16)]

    def group_step(g, carry):
        lv = jnp.where(g < 2, lab_lo, lab_hi)
        goff = (g % 2) * 8
        lab_b = [lv[iota16 * 0 + (goff + r)] for r in range(8)]

        for r in range(8):
            rs = pl.ds(r * 16, 16)
            acc_m[rs] = jnp.full((16,), _NEG, jnp.float32)
            acc_s[rs] = jnp.zeros((16,), jnp.float32)
            acc_t[rs] = jnp.zeros((16,), jnp.float32)
            acc_g[rs] = jnp.zeros((16,), jnp.float32)

        def chunk_step(p, carry2):
            for b in range(NBUF):
                l = p * NBUF + b
                buf, sem = bufs[b], sems[b]
                pltpu.make_async_copy(src(g, l), buf, sem).wait()

                for r in range(8):
                    rs = pl.ds(r * 16, 16)

                    def pass1(j, c, r=r):
                        cm, ct, cg, colv = c
                        v = buf[r, pl.ds(j * 16, 16)]
                        cg = cg + jnp.where(
                            colv == lab_b[r], v, jnp.float32(0.0)
                        )
                        return jnp.maximum(cm, v), ct + v, cg, colv + 16

                    cm, ct, cg, _cv = lax.fori_loop(
                        0,
                        NVREG,
                        pass1,
                        (
                            jnp.full((16,), _NEG, jnp.float32),
                            jnp.zeros((16,), jnp.float32),
                            jnp.zeros((16,), jnp.float32),
                            iota16 + l * CW,
                        ),
                        unroll=8,
                    )
                    m_old = acc_m[rs]
                    m_new = jnp.maximum(m_old, cm)
                    acc_m[rs] = m_new
                    acc_t[rs] = acc_t[rs] + ct
                    acc_g[rs] = acc_g[rs] + cg
                    s_scaled = acc_s[rs] * jnp.exp(m_old - m_new)

                    def pass2(j, acc, r=r, m_new=m_new):
                        v = buf[r, pl.ds(j * 16, 16)]
                        return acc + jnp.exp(v - m_new)

                    acc_s[rs] = lax.fori_loop(
                        0, NVREG, pass2, s_scaled, unroll=8
                    )

                nxt = p * NBUF + b + NBUF

                @pl.when(nxt < SC_CHUNKS)
                def _(buf=buf, sem=sem, nxt=nxt):
                    pltpu.async_copy(src(g, nxt), buf, sem)

                @pl.when(jnp.logical_and(nxt >= SC_CHUNKS, g < GPW - 1))
                def _(buf=buf, sem=sem, nxt=nxt):
                    pltpu.async_copy(src(g + 1, nxt - SC_CHUNKS), buf, sem)

            return carry2

        lax.fori_loop(0, SC_CHUNKS // NBUF, chunk_step, jnp.int32(0))

        base = (row0 + g * 8) * 16
        pltpu.sync_copy(acc_m, m_out.at[pl.ds(base, 128)])
        pltpu.sync_copy(acc_s, s_out.at[pl.ds(base, 128)])
        pltpu.sync_copy(acc_t, t_out.at[pl.ds(base, 128)])
        pltpu.sync_copy(acc_g, g_out.at[pl.ds(base, 128)])
        return carry

    lax.fori_loop(0, GPW, group_step, jnp.int32(0))


def _tc_body(x_ref, lab_ref, m_out, s_out, t_out, g_out, m_ref, s_ref, t_ref, g_ref):
    k = pl.program_id(0)

    @pl.when(k == 0)
    def _init():
        m_ref[...] = jnp.full((B, 1), -jnp.inf, jnp.float32)
        s_ref[...] = jnp.zeros((B, 1), jnp.float32)
        t_ref[...] = jnp.zeros((B, 1), jnp.float32)
        g_ref[...] = jnp.zeros((B, 1), jnp.float32)

    x = x_ref[...]
    ids = jax.lax.broadcasted_iota(jnp.int32, (1, CW), 1) + (
        SC_CHUNKS + k
    ) * CW
    valid = ids < C

    xm = jnp.where(valid, x, -jnp.inf)
    m_prev = m_ref[...]
    m_new = jnp.maximum(m_prev, jnp.max(xm, axis=1, keepdims=True))
    s_blk = jnp.sum(jnp.exp(xm - m_new), axis=1, keepdims=True)
    s_ref[...] = s_ref[...] * jnp.exp(m_prev - m_new) + s_blk
    m_ref[...] = m_new

    t_ref[...] += jnp.sum(jnp.where(valid, x, 0.0), axis=1, keepdims=True)

    lab = lab_ref[...]
    eq = ids == lab
    g_ref[...] += jnp.sum(jnp.where(eq, x, 0.0), axis=1, keepdims=True)

    @pl.when(k == K_TC - 1)
    def _fin():
        m_out[...] = m_ref[...]
        s_out[...] = s_ref[...]
        t_out[...] = t_ref[...]
        g_out[...] = g_ref[...]


def _combine_body(
    lab_ref, ma_ref, sa_ref, ta_ref, ga_ref, mb_ref, sb_ref, tb_ref, gb_ref, out_ref
):
    m16 = ma_ref[...]
    ma = jnp.max(m16, axis=1, keepdims=True)
    sa = jnp.sum(sa_ref[...] * jnp.exp(m16 - ma), axis=1, keepdims=True)
    ta = jnp.sum(ta_ref[...], axis=1, keepdims=True)
    ga = jnp.sum(ga_ref[...], axis=1, keepdims=True)

    mb = mb_ref[...]
    m = jnp.maximum(ma, mb)
    s = sa * jnp.exp(ma - m) + sb_ref[...] * jnp.exp(mb - m)
    t = ta + tb_ref[...]
    g = ga + gb_ref[...]
    lse = m + jnp.log(s)
    lab = lab_ref[...]
    ign = lab == LB_IGNORE
    contrib = LB_NEG * (t - C * lse) + (LB_POS - LB_NEG) * (g - lse)
    contrib = jnp.where(ign, 0.0, contrib)
    n_valid = jnp.sum(jnp.where(ign, 0.0, 1.0))
    out_ref[...] = (-jnp.sum(contrib) / n_valid).reshape(1, 1)


@jax.jit
def kernel(logits, label):
    m_sc, s_sc, t_sc, g_sc = _sc_stats(logits[:, :CS], label)

    lab2 = label.reshape(B, 1)
    stat_spec = pl.BlockSpec((B, 1), lambda k: (0, 0))
    m_tc, s_tc, t_tc, g_tc = pl.pallas_call(
        _tc_body,
        grid=(K_TC,),
        in_specs=[
            pl.BlockSpec((B, CW), lambda k: (0, k + SC_CHUNKS)),
            stat_spec,
        ],
        out_specs=[stat_spec] * 4,
        out_shape=[jax.ShapeDtypeStruct((B, 1), jnp.float32)] * 4,
        scratch_shapes=[pltpu.VMEM((B, 1), jnp.float32)] * 4,
        compiler_params=pltpu.CompilerParams(
            dimension_semantics=("arbitrary",),
        ),
    )(logits, lab2)

    out = pl.pallas_call(
        _combine_body,
        out_shape=jax.ShapeDtypeStruct((1, 1), jnp.float32),
    )(
        lab2,
        m_sc.reshape(B, 16),
        s_sc.reshape(B, 16),
        t_sc.reshape(B, 16),
        g_sc.reshape(B, 16),
        m_tc,
        s_tc,
        t_tc,
        g_tc,
    )
    return out[0, 0]


# speedup vs baseline: 1.3921x; 1.3921x over previous
import functools

import jax
import jax.numpy as jnp
from jax.experimental import pallas as pl
from jax.experimental.pallas import tpu as pltpu

LB_POS = 0.9
LB_NEG = 0.005
LB_IGNORE = 255

B = 1024
C = 100000
BC = 2048
K = (C + BC - 1) // BC


def _body(x_ref, lab_ref, out_ref, m_ref, s_ref, t_ref, g_ref):
    k = pl.program_id(0)

    @pl.when(k == 0)
    def _init():
        m_ref[...] = jnp.full((B, 1), -jnp.inf, jnp.float32)
        s_ref[...] = jnp.zeros((B, 1), jnp.float32)
        t_ref[...] = jnp.zeros((B, 1), jnp.float32)
        g_ref[...] = jnp.zeros((B, 1), jnp.float32)

    x = x_ref[...]
    ids = jax.lax.broadcasted_iota(jnp.int32, (1, BC), 1) + k * BC
    valid = ids < C

    xm = jnp.where(valid, x, -jnp.inf)
    m_prev = m_ref[...]
    m_new = jnp.maximum(m_prev, jnp.max(xm, axis=1, keepdims=True))
    s_blk = jnp.sum(jnp.exp(xm - m_new), axis=1, keepdims=True)
    s_ref[...] = s_ref[...] * jnp.exp(m_prev - m_new) + s_blk
    m_ref[...] = m_new

    t_ref[...] += jnp.sum(jnp.where(valid, x, 0.0), axis=1, keepdims=True)

    lab = lab_ref[...]
    eq = ids == lab
    g_ref[...] += jnp.sum(jnp.where(eq, x, 0.0), axis=1, keepdims=True)

    @pl.when(k == K - 1)
    def _fin():
        lse = m_ref[...] + jnp.log(s_ref[...])
        ign = lab == LB_IGNORE
        contrib = LB_NEG * (t_ref[...] - C * lse) + (LB_POS - LB_NEG) * (
            g_ref[...] - lse
        )
        contrib = jnp.where(ign, 0.0, contrib)
        n_valid = jnp.sum(jnp.where(ign, 0.0, 1.0))
        out_ref[...] = (-jnp.sum(contrib) / n_valid).reshape(1, 1)


@jax.jit
def kernel(logits, label):
    out = pl.pallas_call(
        _body,
        grid=(K,),
        in_specs=[
            pl.BlockSpec((B, BC), lambda k: (0, k)),
            pl.BlockSpec((B, 1), lambda k: (0, 0)),
        ],
        out_specs=pl.BlockSpec((1, 1), lambda k: (0, 0)),
        out_shape=jax.ShapeDtypeStruct((1, 1), jnp.float32),
        scratch_shapes=[pltpu.VMEM((B, 1), jnp.float32)] * 4,
        compiler_params=pltpu.CompilerParams(
            dimension_semantics=("arbitrary",),
        ),
    )(logits, label.reshape(B, 1))
    return out[0, 0]
